# baseline (speedup 1.0000x reference)
import functools

import jax
import jax.numpy as jnp
from jax import lax
from jax.experimental import pallas as pl
from jax.experimental.pallas import tpu as pltpu
from jax.experimental.pallas import tpu_sc as plsc

BATCH = 16384
HID = 64
N_LOOKUPS = 3
TOTAL = N_LOOKUPS * BATCH

_INFO = plsc.get_sparse_core_info()
_NC = _INFO.num_cores
_NS = _INFO.num_subcores
_NW = _NC * _NS
_CHUNK = 128
_B_PER_W = TOTAL // _NW
_N_CHUNKS = _B_PER_W // _CHUNK


def _make_gather():
    mesh = plsc.VectorSubcoreMesh(core_axis_name="c", subcore_axis_name="s")

    @functools.partial(
        pl.kernel,
        mesh=mesh,
        compiler_params=pltpu.CompilerParams(use_tc_tiling_on_sc=False),
        out_type=jax.ShapeDtypeStruct((TOTAL, HID), jnp.float32),
        scratch_types=[
            pltpu.VMEM((_N_CHUNKS, _CHUNK), jnp.int32),
            pltpu.VMEM((_B_PER_W, HID), jnp.float32),
            pltpu.SemaphoreType.DMA,
        ],
    )
    def gather_kernel(table_hbm, idx_hbm, out_hbm, idx_v, rows_v, sem):
        wid = lax.axis_index("s") * _NC + lax.axis_index("c")
        base = wid * _B_PER_W
        pltpu.sync_copy(idx_hbm.at[wid], idx_v)
        copies = []
        for j in range(_N_CHUNKS):
            copies.append(
                pltpu.async_copy(
                    table_hbm.at[idx_v.at[j]],
                    rows_v.at[pl.ds(j * _CHUNK, _CHUNK)],
                    sem,
                )
            )
        for c in copies:
            c.wait()
        pltpu.sync_copy(rows_v, out_hbm.at[pl.ds(base, _B_PER_W)])

    return gather_kernel


_gather = _make_gather()

_NB = 8
_BM = BATCH // _NB


def _dense_body(g3_ref, r_ref, i_ref, u_ref, w_ref, b_ref, eye_ref,
                p_ref, h_ref):
    eye = eye_ref[...]
    dn = (((1,), (1,)), ((), ()))
    hi = lax.Precision.HIGHEST
    for j in range(N_LOOKUPS):
        xj = g3_ref[:, j, :]
        p_ref[j] = lax.dot_general(eye, xj, dn, precision=hi,
                                   preferred_element_type=jnp.float32)
    w = w_ref[...]
    acc = lax.dot_general(w[:, 0:HID], r_ref[...], dn, precision=hi,
                          preferred_element_type=jnp.float32)
    acc += lax.dot_general(w[:, HID:2 * HID], i_ref[...], dn, precision=hi,
                           preferred_element_type=jnp.float32)
    acc += lax.dot_general(w[:, 2 * HID:3 * HID], u_ref[...], dn,
                           precision=hi,
                           preferred_element_type=jnp.float32)
    h_ref[...] = jnp.tanh(acc + b_ref[...])


def kernel(userId, itemId, rate, table, W, b):
    flat_idx = jnp.concatenate(
        [rate.astype(jnp.int32), itemId.astype(jnp.int32),
         userId.astype(jnp.int32)]
    ).reshape(_NW, _N_CHUNKS, _CHUNK)

    gathered = _gather(table, flat_idx)
    gathered3 = gathered.reshape(BATCH, N_LOOKUPS, HID)

    P, H = pl.pallas_call(
        _dense_body,
        grid=(_NB,),
        in_specs=[
            pl.BlockSpec((_BM, N_LOOKUPS, HID), lambda i: (i, 0, 0)),
            pl.BlockSpec((_BM, HID), lambda i: (i, 0)),
            pl.BlockSpec((_BM, HID), lambda i: (i + _NB, 0)),
            pl.BlockSpec((_BM, HID), lambda i: (i + 2 * _NB, 0)),
            pl.BlockSpec((HID, N_LOOKUPS * HID), lambda i: (0, 0)),
            pl.BlockSpec((HID, 1), lambda i: (0, 0)),
            pl.BlockSpec((HID, HID), lambda i: (0, 0)),
        ],
        out_specs=[
            pl.BlockSpec((N_LOOKUPS, HID, _BM), lambda i: (0, 0, i)),
            pl.BlockSpec((HID, _BM), lambda i: (0, i)),
        ],
        out_shape=[
            jax.ShapeDtypeStruct((N_LOOKUPS, HID, BATCH), jnp.float32),
            jax.ShapeDtypeStruct((HID, BATCH), jnp.float32),
        ],
    )(gathered3, gathered, gathered, gathered, W, b.reshape(HID, 1),
      jnp.eye(HID, dtype=jnp.float32))

    output_list = jnp.transpose(P, (2, 0, 1))
    hidden = H.T
    return (output_list, hidden)


# speedup vs baseline: 1.0583x; 1.0583x over previous
import functools

import jax
import jax.numpy as jnp
from jax import lax
from jax.experimental import pallas as pl
from jax.experimental.pallas import tpu as pltpu
from jax.experimental.pallas import tpu_sc as plsc

BATCH = 16384
HID = 64
N_LOOKUPS = 3
TOTAL = N_LOOKUPS * BATCH

_INFO = plsc.get_sparse_core_info()
_NC = _INFO.num_cores
_NS = _INFO.num_subcores
_NW = _NC * _NS
_CHUNK = 128
_B_PER_W = TOTAL // _NW
_N_CHUNKS = _B_PER_W // _CHUNK


def _make_gather():
    mesh = plsc.VectorSubcoreMesh(core_axis_name="c", subcore_axis_name="s")

    @functools.partial(
        pl.kernel,
        mesh=mesh,
        compiler_params=pltpu.CompilerParams(use_tc_tiling_on_sc=False),
        out_type=jax.ShapeDtypeStruct((TOTAL, HID), jnp.float32),
        scratch_types=[
            pltpu.VMEM((_N_CHUNKS, _CHUNK), jnp.int32),
            pltpu.VMEM((_B_PER_W, HID), jnp.float32),
            pltpu.SemaphoreType.DMA,
        ],
    )
    def gather_kernel(table_hbm, idx_hbm, out_hbm, idx_v, rows_v, sem):
        wid = lax.axis_index("s") * _NC + lax.axis_index("c")
        base = wid * _B_PER_W
        pltpu.sync_copy(idx_hbm.at[wid], idx_v)
        copies = []
        for j in range(_N_CHUNKS):
            copies.append(
                pltpu.async_copy(
                    table_hbm.at[idx_v.at[j]],
                    rows_v.at[pl.ds(j * _CHUNK, _CHUNK)],
                    sem,
                )
            )
        for c in copies:
            c.wait()
        pltpu.sync_copy(rows_v, out_hbm.at[pl.ds(base, _B_PER_W)])

    return gather_kernel


_gather = _make_gather()

_NB = 8
_BM = BATCH // _NB


def _hidden_body(r_ref, i_ref, u_ref, w_ref, b_ref, o_ref):
    w = w_ref[...]
    dn = (((1,), (1,)), ((), ()))
    acc = lax.dot_general(r_ref[...], w[:, 0:HID], dn,
                          preferred_element_type=jnp.float32)
    acc += lax.dot_general(i_ref[...], w[:, HID:2 * HID], dn,
                           preferred_element_type=jnp.float32)
    acc += lax.dot_general(u_ref[...], w[:, 2 * HID:3 * HID], dn,
                           preferred_element_type=jnp.float32)
    o_ref[...] = jnp.tanh(acc + b_ref[...])


def kernel(userId, itemId, rate, table, W, b):
    flat_idx = jnp.concatenate(
        [rate.astype(jnp.int32), itemId.astype(jnp.int32),
         userId.astype(jnp.int32)]
    ).reshape(_NW, _N_CHUNKS, _CHUNK)

    gathered = _gather(table, flat_idx)

    hidden = pl.pallas_call(
        _hidden_body,
        grid=(_NB,),
        in_specs=[
            pl.BlockSpec((_BM, HID), lambda i: (i, 0)),
            pl.BlockSpec((_BM, HID), lambda i: (i + _NB, 0)),
            pl.BlockSpec((_BM, HID), lambda i: (i + 2 * _NB, 0)),
            pl.BlockSpec((HID, N_LOOKUPS * HID), lambda i: (0, 0)),
            pl.BlockSpec((1, HID), lambda i: (0, 0)),
        ],
        out_specs=pl.BlockSpec((_BM, HID), lambda i: (i, 0)),
        out_shape=jax.ShapeDtypeStruct((BATCH, HID), jnp.float32),
    )(gathered, gathered, gathered, W, b.reshape(1, HID))

    output_list = gathered.reshape(BATCH, N_LOOKUPS, HID)
    return (output_list, hidden)
